# Initial kernel scaffold; baseline (speedup 1.0000x reference)
import jax
import jax.numpy as jnp
from jax.experimental import pallas as pl


def kernel(x, edge_index, batch, W_self, b_self, W_neigh, b_neigh, W_fp):
    raise NotImplementedError("write your pallas kernel here")


# speedup vs baseline: 3.5422x; 3.5422x over previous
import functools

import jax
import jax.numpy as jnp
from jax import lax
from jax.experimental import pallas as pl
from jax.experimental.pallas import tpu as pltpu
from jax.experimental.pallas import tpu_sc as plsc

N = 10000
E = 160000
DH = 256
DHALF = 128
FPD = 128
NGR = 64
LAYERS = 3

NTILES = 16
EPT = E // NTILES
CH = 80
NCH = EPT // CH
RPT = 624
TAIL = N - NTILES * RPT
ZR = 24

ROWBLK = 1000
NROWBLK = N // ROWBLK


def _sc_body(h_lo, h_hi, col_r, row_r, out_lo, out_hi,
             colv, rowv, gbuf, zbuf, acc, sem):
    c = lax.axis_index("c")
    s = lax.axis_index("s")

    def zstore(j, _):
        r = j // (DHALF // 16)
        k = j - r * (DHALF // 16)
        zbuf[r, pl.ds(k * 16, 16)] = jnp.zeros((16,), jnp.float32)
        return 0
    lax.fori_loop(0, ZR * (DHALF // 16), zstore, 0)

    def zcopy(z, _):
        pltpu.sync_copy(zbuf, acc.at[pl.ds(s * RPT + z * ZR, ZR)])
        return 0
    lax.fori_loop(0, RPT // ZR, zcopy, 0)

    @pl.when(s == 0)
    def _():
        pltpu.sync_copy(zbuf.at[pl.ds(0, TAIL)],
                        acc.at[pl.ds(NTILES * RPT, TAIL)])

    pltpu.sync_copy(col_r.at[s], colv)
    pltpu.sync_copy(row_r.at[s], rowv)

    plsc.subcore_barrier()

    def run(h_hbm, out_hbm):
        def chunk(i, _):
            pltpu.async_copy(h_hbm.at[colv.at[i]], gbuf, sem).wait()
            pltpu.sync_copy(gbuf, acc.at[rowv.at[i]], add=True)
            return 0
        lax.fori_loop(0, NCH, chunk, 0)
        plsc.subcore_barrier()
        pltpu.sync_copy(acc.at[pl.ds(s * RPT, RPT)],
                        out_hbm.at[pl.ds(s * RPT, RPT)])

        @pl.when(s == 0)
        def _():
            pltpu.sync_copy(acc.at[pl.ds(NTILES * RPT, TAIL)],
                            out_hbm.at[pl.ds(NTILES * RPT, TAIL)])

    @pl.when(c == 0)
    def _():
        run(h_lo, out_lo)

    @pl.when(c == 1)
    def _():
        run(h_hi, out_hi)


def _sc_aggregate(h_lo, h_hi, col_r, row_r):
    mesh = plsc.VectorSubcoreMesh(core_axis_name="c", subcore_axis_name="s")
    return pl.kernel(
        _sc_body,
        out_type=(
            jax.ShapeDtypeStruct((N, DHALF), jnp.float32),
            jax.ShapeDtypeStruct((N, DHALF), jnp.float32),
        ),
        mesh=mesh,
        scratch_types=[
            pltpu.VMEM((NCH, CH), jnp.int32),
            pltpu.VMEM((NCH, CH), jnp.int32),
            pltpu.VMEM((CH, DHALF), jnp.float32),
            pltpu.VMEM((ZR, DHALF), jnp.float32),
            pltpu.VMEM_SHARED((N, DHALF), jnp.float32),
            pltpu.SemaphoreType.DMA,
        ],
    )(h_lo, h_hi, col_r, row_r)


def _tc_layer_body(hlo_ref, hhi_ref, nlo_ref, nhi_ref, ws_ref, wn_ref,
                   b_ref, wfp_ref, batch_ref, olo_ref, ohi_ref, fp_ref):
    i = pl.program_id(0)
    bf = jnp.bfloat16
    h = jnp.concatenate([hlo_ref[...], hhi_ref[...]], axis=1).astype(bf)
    nb = jnp.concatenate([nlo_ref[...], nhi_ref[...]], axis=1).astype(bf)
    z = (jnp.dot(h, ws_ref[...].astype(bf), preferred_element_type=jnp.float32)
         + jnp.dot(nb, wn_ref[...].astype(bf), preferred_element_type=jnp.float32)
         + b_ref[...])
    hn = jnp.tanh(z)
    logits = jnp.dot(hn.astype(bf), wfp_ref[...].astype(bf),
                     preferred_element_type=jnp.float32)
    m = jnp.max(logits, axis=1, keepdims=True)
    e = jnp.exp(logits - m)
    p = e / jnp.sum(e, axis=1, keepdims=True)

    b = batch_ref[0, 0, :]
    onehot = (b[:, None] == lax.broadcasted_iota(jnp.int32, (ROWBLK, NGR), 1)
              ).astype(jnp.float32)
    part = lax.dot_general(onehot, p, (((0,), (0,)), ((), ())),
                           preferred_element_type=jnp.float32)

    olo_ref[...] = hn[:, :DHALF]
    ohi_ref[...] = hn[:, DHALF:]

    @pl.when(i == 0)
    def _():
        fp_ref[...] = jnp.zeros_like(fp_ref)
    fp_ref[...] += part


def _tc_layer(h_lo, h_hi, n_lo, n_hi, ws, wn, bias, wfp, batch_r):
    half_spec = pl.BlockSpec((ROWBLK, DHALF), lambda i: (i, 0))
    full = lambda shape: pl.BlockSpec(shape, lambda i: tuple(0 for _ in shape))
    return pl.pallas_call(
        _tc_layer_body,
        grid=(NROWBLK,),
        in_specs=[
            half_spec, half_spec, half_spec, half_spec,
            full((DH, DH)), full((DH, DH)), full((1, DH)), full((DH, FPD)),
            pl.BlockSpec((1, 1, ROWBLK), lambda i: (i, 0, 0)),
        ],
        out_specs=[half_spec, half_spec, full((NGR, FPD))],
        out_shape=(
            jax.ShapeDtypeStruct((N, DHALF), jnp.float32),
            jax.ShapeDtypeStruct((N, DHALF), jnp.float32),
            jax.ShapeDtypeStruct((NGR, FPD), jnp.float32),
        ),
    )(h_lo, h_hi, n_lo, n_hi, ws, wn, bias, wfp, batch_r)


def kernel(x, edge_index, batch, W_self, b_self, W_neigh, b_neigh, W_fp):
    row = edge_index[0]
    col = edge_index[1]
    order = jnp.argsort(row, stable=True)
    col_r = col[order].reshape(NTILES, NCH, CH)
    row_r = row[order].reshape(NTILES, NCH, CH)
    batch_r = batch.reshape(NROWBLK, 1, ROWBLK)

    h_lo = x[:, :DHALF]
    h_hi = x[:, DHALF:]

    fps = []
    for l in range(LAYERS):
        n_lo, n_hi = _sc_aggregate(h_lo, h_hi, col_r, row_r)
        ws = W_self[l].T
        wn = W_neigh[l].T
        bias = (b_self[l] + b_neigh[l]).reshape(1, DH)
        wfp = W_fp[l].T
        h_lo, h_hi, fp = _tc_layer(h_lo, h_hi, n_lo, n_hi,
                                   ws, wn, bias, wfp, batch_r)
        fps.append(fp)
    return fps[0] + fps[1] + fps[2]


# speedup vs baseline: 4.0968x; 1.1566x over previous
import functools

import jax
import jax.numpy as jnp
from jax import lax
from jax.experimental import pallas as pl
from jax.experimental.pallas import tpu as pltpu
from jax.experimental.pallas import tpu_sc as plsc

N = 10000
E = 160000
DH = 256
DHALF = 128
FPD = 128
NGR = 64
LAYERS = 3

NTILES = 16
EPT = E // NTILES
CH = 40
NCH = EPT // CH
IG = 50
NGRP = NCH // IG
NPAIR = IG // 2
RPT = 624
TAIL = N - NTILES * RPT
ZR = 8

ROWBLK = 1000
NROWBLK = N // ROWBLK


def _sc_body(h_lo, h_hi, col_r, row_r, out_lo, out_hi,
             colv, rowv, gbuf, zbuf, acc, sem_a, sem_b, sem_idx):
    c = lax.axis_index("c")
    s = lax.axis_index("s")

    def zstore(j, _):
        r = j // (DHALF // 16)
        k = j - r * (DHALF // 16)
        zbuf[r, pl.ds(k * 16, 16)] = jnp.zeros((16,), jnp.float32)
        return 0
    lax.fori_loop(0, ZR * (DHALF // 16), zstore, 0)

    def zcopy(z, _):
        pltpu.sync_copy(zbuf, acc.at[pl.ds(s * RPT + z * ZR, ZR)])
        return 0
    lax.fori_loop(0, RPT // ZR, zcopy, 0)

    @pl.when(s == 0)
    def _():
        pltpu.sync_copy(zbuf.at[pl.ds(0, TAIL)],
                        acc.at[pl.ds(NTILES * RPT, TAIL)])

    pltpu.sync_copy(col_r.at[s, 0], colv.at[0])
    pltpu.sync_copy(row_r.at[s, 0], rowv.at[0])

    plsc.subcore_barrier()

    def run(h_hbm, out_hbm):
        pltpu.async_copy(h_hbm.at[colv.at[0, 0]], gbuf.at[0], sem_a)
        pltpu.async_copy(h_hbm.at[colv.at[0, 1]], gbuf.at[1], sem_b)

        for g in range(NGRP):
            gb = g % 2
            if g + 1 < NGRP:
                pltpu.async_copy(col_r.at[s, g + 1], colv.at[1 - gb], sem_idx)
                pltpu.async_copy(row_r.at[s, g + 1], rowv.at[1 - gb], sem_idx)

            def pair(t, _, gb=gb):
                j = 2 * t
                pltpu.make_async_copy(h_hbm.at[colv.at[gb, j]],
                                      gbuf.at[0], sem_a).wait()
                pltpu.sync_copy(gbuf.at[0], acc.at[rowv.at[gb, j]], add=True)

                @pl.when(j + 2 < IG)
                def _():
                    pltpu.async_copy(h_hbm.at[colv.at[gb, j + 2]],
                                     gbuf.at[0], sem_a)

                pltpu.make_async_copy(h_hbm.at[colv.at[gb, j + 1]],
                                      gbuf.at[1], sem_b).wait()
                pltpu.sync_copy(gbuf.at[1], acc.at[rowv.at[gb, j + 1]],
                                add=True)

                @pl.when(j + 3 < IG)
                def _():
                    pltpu.async_copy(h_hbm.at[colv.at[gb, j + 3]],
                                     gbuf.at[1], sem_b)
                return 0
            lax.fori_loop(0, NPAIR, pair, 0)

            if g + 1 < NGRP:
                pltpu.make_async_copy(col_r.at[s, g + 1], colv.at[1 - gb],
                                      sem_idx).wait()
                pltpu.make_async_copy(row_r.at[s, g + 1], rowv.at[1 - gb],
                                      sem_idx).wait()
                pltpu.async_copy(h_hbm.at[colv.at[1 - gb, 0]],
                                 gbuf.at[0], sem_a)
                pltpu.async_copy(h_hbm.at[colv.at[1 - gb, 1]],
                                 gbuf.at[1], sem_b)

        plsc.subcore_barrier()
        pltpu.sync_copy(acc.at[pl.ds(s * RPT, RPT)],
                        out_hbm.at[pl.ds(s * RPT, RPT)])

        @pl.when(s == 0)
        def _():
            pltpu.sync_copy(acc.at[pl.ds(NTILES * RPT, TAIL)],
                            out_hbm.at[pl.ds(NTILES * RPT, TAIL)])

    @pl.when(c == 0)
    def _():
        run(h_lo, out_lo)

    @pl.when(c == 1)
    def _():
        run(h_hi, out_hi)


def _sc_aggregate(h_lo, h_hi, col_r, row_r):
    mesh = plsc.VectorSubcoreMesh(core_axis_name="c", subcore_axis_name="s")
    return pl.kernel(
        _sc_body,
        out_type=(
            jax.ShapeDtypeStruct((N, DHALF), jnp.float32),
            jax.ShapeDtypeStruct((N, DHALF), jnp.float32),
        ),
        mesh=mesh,
        scratch_types=[
            pltpu.VMEM((2, IG, CH), jnp.int32),
            pltpu.VMEM((2, IG, CH), jnp.int32),
            pltpu.VMEM((2, CH, DHALF), jnp.float32),
            pltpu.VMEM((ZR, DHALF), jnp.float32),
            pltpu.VMEM_SHARED((N, DHALF), jnp.float32),
            pltpu.SemaphoreType.DMA,
            pltpu.SemaphoreType.DMA,
            pltpu.SemaphoreType.DMA,
        ],
    )(h_lo, h_hi, col_r, row_r)


def _tc_layer_body(hlo_ref, hhi_ref, nlo_ref, nhi_ref, ws_ref, wn_ref,
                   b_ref, wfp_ref, batch_ref, olo_ref, ohi_ref, fp_ref):
    i = pl.program_id(0)
    bf = jnp.bfloat16
    h = jnp.concatenate([hlo_ref[...], hhi_ref[...]], axis=1).astype(bf)
    nb = jnp.concatenate([nlo_ref[...], nhi_ref[...]], axis=1).astype(bf)
    z = (jnp.dot(h, ws_ref[...].astype(bf), preferred_element_type=jnp.float32)
         + jnp.dot(nb, wn_ref[...].astype(bf), preferred_element_type=jnp.float32)
         + b_ref[...])
    hn = jnp.tanh(z)
    logits = jnp.dot(hn.astype(bf), wfp_ref[...].astype(bf),
                     preferred_element_type=jnp.float32)
    m = jnp.max(logits, axis=1, keepdims=True)
    e = jnp.exp(logits - m)
    p = e / jnp.sum(e, axis=1, keepdims=True)

    b = batch_ref[0, 0, :]
    onehot = (b[:, None] == lax.broadcasted_iota(jnp.int32, (ROWBLK, NGR), 1)
              ).astype(jnp.float32)
    part = lax.dot_general(onehot, p, (((0,), (0,)), ((), ())),
                           preferred_element_type=jnp.float32)

    olo_ref[...] = hn[:, :DHALF]
    ohi_ref[...] = hn[:, DHALF:]

    @pl.when(i == 0)
    def _():
        fp_ref[...] = jnp.zeros_like(fp_ref)
    fp_ref[...] += part


def _tc_layer(h_lo, h_hi, n_lo, n_hi, ws, wn, bias, wfp, batch_r):
    half_spec = pl.BlockSpec((ROWBLK, DHALF), lambda i: (i, 0))
    full = lambda shape: pl.BlockSpec(shape, lambda i: tuple(0 for _ in shape))
    return pl.pallas_call(
        _tc_layer_body,
        grid=(NROWBLK,),
        in_specs=[
            half_spec, half_spec, half_spec, half_spec,
            full((DH, DH)), full((DH, DH)), full((1, DH)), full((DH, FPD)),
            pl.BlockSpec((1, 1, ROWBLK), lambda i: (i, 0, 0)),
        ],
        out_specs=[half_spec, half_spec, full((NGR, FPD))],
        out_shape=(
            jax.ShapeDtypeStruct((N, DHALF), jnp.float32),
            jax.ShapeDtypeStruct((N, DHALF), jnp.float32),
            jax.ShapeDtypeStruct((NGR, FPD), jnp.float32),
        ),
    )(h_lo, h_hi, n_lo, n_hi, ws, wn, bias, wfp, batch_r)


def kernel(x, edge_index, batch, W_self, b_self, W_neigh, b_neigh, W_fp):
    row = edge_index[0]
    col = edge_index[1]
    order = jnp.argsort(row, stable=True)
    col_r = col[order].reshape(NTILES, NGRP, IG, CH)
    row_r = row[order].reshape(NTILES, NGRP, IG, CH)
    batch_r = batch.reshape(NROWBLK, 1, ROWBLK)

    h_lo = x[:, :DHALF]
    h_hi = x[:, DHALF:]

    fps = []
    for l in range(LAYERS):
        n_lo, n_hi = _sc_aggregate(h_lo, h_hi, col_r, row_r)
        ws = W_self[l].T
        wn = W_neigh[l].T
        bias = (b_self[l] + b_neigh[l]).reshape(1, DH)
        wfp = W_fp[l].T
        h_lo, h_hi, fp = _tc_layer(h_lo, h_hi, n_lo, n_hi,
                                   ws, wn, bias, wfp, batch_r)
        fps.append(fp)
    return fps[0] + fps[1] + fps[2]
